# Initial kernel scaffold; baseline (speedup 1.0000x reference)
import jax
import jax.numpy as jnp
from jax.experimental import pallas as pl


def kernel(x, W_enc, W_dec, b_pre):
    raise NotImplementedError("write your pallas kernel here")


# speedup vs baseline: 4.4678x; 4.4678x over previous
import functools

import jax
import jax.numpy as jnp
from jax.experimental import pallas as pl
from jax.experimental.pallas import tpu as pltpu

K = 32


def _to_key(p):
    s = jax.lax.bitcast_convert_type(p, jnp.int32)
    flip = jnp.where(s < 0, jnp.int32(0x7FFFFFFF), jnp.int32(0))
    return s ^ flip


def _encode_kernel(x_ref, b_ref, w_ref, z_ref, nrm_ref, mu_ref,
                   keys_ref, xn_ref, thr_ref, *, nj, lb, k):
    j = pl.program_id(1)

    @pl.when(j == 0)
    def _preprocess():
        xs = x_ref[...] - b_ref[...]
        mu = jnp.mean(xs, axis=1, keepdims=True)
        c = xs - mu
        nrm = jnp.maximum(jnp.sqrt(jnp.sum(c * c, axis=1, keepdims=True)),
                          1e-8)
        xn_ref[...] = (c / nrm).astype(jnp.bfloat16)
        nrm_ref[...] = nrm
        mu_ref[...] = mu

    @pl.when(j < nj)
    def _matmul():
        p = jax.lax.dot_general(
            xn_ref[...], w_ref[...],
            dimension_numbers=(((1,), (1,)), ((), ())),
            preferred_element_type=jnp.float32)
        keys_ref[:, pl.ds(j * lb, lb)] = _to_key(p)

    @pl.when(j == nj - 1)
    def _bisect():
        rb = thr_ref.shape[0]

        def body(_, carry):
            lo, hi = carry
            mid = (lo >> 1) + (hi >> 1) + ((lo | hi) & 1)
            cnt = jnp.sum((keys_ref[...] >= mid).astype(jnp.int32),
                          axis=1, keepdims=True)
            ge = cnt >= k
            return jnp.where(ge, mid, lo), jnp.where(ge, hi, mid)

        lo0 = jnp.full((rb, 1), -2147483648, jnp.int32)
        hi0 = jnp.full((rb, 1), 2147483647, jnp.int32)
        lo, _ = jax.lax.fori_loop(0, 32, body, (lo0, hi0))
        thr_ref[...] = jnp.maximum(lo, 0)

    @pl.when(j >= nj)
    def _write_z():
        jj = j - nj
        keys = keys_ref[:, pl.ds(jj * lb, lb)]
        sel = keys >= thr_ref[...]
        z_ref[...] = jnp.where(sel, jax.lax.bitcast_convert_type(
            jnp.maximum(keys, 0), jnp.float32), 0.0)


def _decode_kernel(z_ref, w_ref, nrm_ref, mu_ref, b_ref, out_ref, acc_ref,
                   *, nk):
    kk = pl.program_id(1)

    @pl.when(kk == 0)
    def _init():
        acc_ref[...] = jnp.zeros_like(acc_ref)

    acc_ref[...] += jax.lax.dot_general(
        z_ref[...].astype(jnp.bfloat16), w_ref[...],
        dimension_numbers=(((1,), (1,)), ((), ())),
        preferred_element_type=jnp.float32)

    @pl.when(kk == nk - 1)
    def _restore():
        out_ref[...] = acc_ref[...] * nrm_ref[...] + mu_ref[...] + b_ref[...]


@jax.jit
def kernel(x, W_enc, W_dec, b_pre):
    tokens, d = x.shape
    latent = W_enc.shape[0]
    rb = min(256, tokens)
    lb = min(1024, latent)
    ni = tokens // rb
    nj = latent // lb
    b2 = b_pre.reshape(1, d)
    w_enc_bf = W_enc.astype(jnp.bfloat16)
    w_dec_bf = W_dec.astype(jnp.bfloat16)

    z, nrm, mu = pl.pallas_call(
        functools.partial(_encode_kernel, nj=nj, lb=lb, k=K),
        grid=(ni, 2 * nj),
        in_specs=[
            pl.BlockSpec((rb, d), lambda i, j: (i, 0)),
            pl.BlockSpec((1, d), lambda i, j: (0, 0)),
            pl.BlockSpec((lb, d), lambda i, j, _nj=nj: (jnp.minimum(j, _nj - 1), 0)),
        ],
        out_specs=[
            pl.BlockSpec((rb, lb), lambda i, j, _nj=nj: (i, jnp.maximum(j - _nj, 0))),
            pl.BlockSpec((rb, 1), lambda i, j: (i, 0)),
            pl.BlockSpec((rb, 1), lambda i, j: (i, 0)),
        ],
        out_shape=[
            jax.ShapeDtypeStruct((tokens, latent), jnp.float32),
            jax.ShapeDtypeStruct((tokens, 1), jnp.float32),
            jax.ShapeDtypeStruct((tokens, 1), jnp.float32),
        ],
        scratch_shapes=[
            pltpu.VMEM((rb, latent), jnp.int32),
            pltpu.VMEM((rb, d), jnp.bfloat16),
            pltpu.VMEM((rb, 1), jnp.int32),
        ],
    )(x, b2, w_enc_bf)

    lbd = min(2048, latent)
    nk = latent // lbd
    x_hat = pl.pallas_call(
        functools.partial(_decode_kernel, nk=nk),
        grid=(ni, nk),
        in_specs=[
            pl.BlockSpec((rb, lbd), lambda i, kk: (i, kk)),
            pl.BlockSpec((d, lbd), lambda i, kk: (0, kk)),
            pl.BlockSpec((rb, 1), lambda i, kk: (i, 0)),
            pl.BlockSpec((rb, 1), lambda i, kk: (i, 0)),
            pl.BlockSpec((1, d), lambda i, kk: (0, 0)),
        ],
        out_specs=pl.BlockSpec((rb, d), lambda i, kk: (i, 0)),
        out_shape=jax.ShapeDtypeStruct((tokens, d), jnp.float32),
        scratch_shapes=[pltpu.VMEM((rb, d), jnp.float32)],
    )(z, w_dec_bf, nrm, mu, b2)

    return (x_hat, z)


# speedup vs baseline: 4.5514x; 1.0187x over previous
import functools

import jax
import jax.numpy as jnp
from jax.experimental import pallas as pl
from jax.experimental.pallas import tpu as pltpu

K = 32


def _to_key(p):
    s = jax.lax.bitcast_convert_type(p, jnp.int32)
    flip = jnp.where(s < 0, jnp.int32(0x7FFFFFFF), jnp.int32(0))
    return s ^ flip


def _sae_kernel(x_ref, b_ref, we_ref, wd_ref, z_ref, xh_ref,
                keys_ref, xn_ref, thr_ref, nrm_ref, mu_ref, acc_ref,
                *, nj, lb, k):
    j = pl.program_id(1)

    @pl.when(j == 0)
    def _preprocess():
        xs = x_ref[...] - b_ref[...]
        mu = jnp.mean(xs, axis=1, keepdims=True)
        c = xs - mu
        nrm = jnp.maximum(jnp.sqrt(jnp.sum(c * c, axis=1, keepdims=True)),
                          1e-8)
        xn_ref[...] = (c / nrm).astype(jnp.bfloat16)
        nrm_ref[...] = nrm
        mu_ref[...] = mu

    @pl.when(j < nj)
    def _encode():
        p = jax.lax.dot_general(
            xn_ref[...], we_ref[...],
            dimension_numbers=(((1,), (1,)), ((), ())),
            preferred_element_type=jnp.float32)
        keys_ref[:, pl.ds(j * lb, lb)] = _to_key(p)

    @pl.when(j == nj - 1)
    def _bisect():
        rb = thr_ref.shape[0]

        def body(_, carry):
            lo, hi = carry
            mid = (lo >> 1) + (hi >> 1) + ((lo | hi) & 1)
            cnt = jnp.sum((keys_ref[...] >= mid).astype(jnp.int32),
                          axis=1, keepdims=True)
            ge = cnt >= k
            return jnp.where(ge, mid, lo), jnp.where(ge, hi, mid)

        lo0 = jnp.full((rb, 1), -2147483648, jnp.int32)
        hi0 = jnp.full((rb, 1), 2147483647, jnp.int32)
        lo, _ = jax.lax.fori_loop(0, 32, body, (lo0, hi0))
        thr_ref[...] = jnp.maximum(lo, 0)

    @pl.when(j >= nj)
    def _decode():
        jj = j - nj
        keys = keys_ref[:, pl.ds(jj * lb, lb)]
        sel = keys >= thr_ref[...]
        zblk = jnp.where(sel, jax.lax.bitcast_convert_type(
            jnp.maximum(keys, 0), jnp.float32), 0.0)
        z_ref[...] = zblk

        @pl.when(j == nj)
        def _init_acc():
            acc_ref[...] = jnp.zeros_like(acc_ref)

        acc_ref[...] += jax.lax.dot_general(
            zblk.astype(jnp.bfloat16), wd_ref[...],
            dimension_numbers=(((1,), (1,)), ((), ())),
            preferred_element_type=jnp.float32)

        @pl.when(j == 2 * nj - 1)
        def _restore():
            xh_ref[...] = acc_ref[...] * nrm_ref[...] + mu_ref[...] + b_ref[...]


@jax.jit
def kernel(x, W_enc, W_dec, b_pre):
    tokens, d = x.shape
    latent = W_enc.shape[0]
    rb = min(256, tokens)
    lb = min(1024, latent)
    ni = tokens // rb
    nj = latent // lb
    b2 = b_pre.reshape(1, d)
    w_enc_bf = W_enc.astype(jnp.bfloat16)
    w_dec_bf = W_dec.astype(jnp.bfloat16)

    z, x_hat = pl.pallas_call(
        functools.partial(_sae_kernel, nj=nj, lb=lb, k=K),
        grid=(ni, 2 * nj),
        in_specs=[
            pl.BlockSpec((rb, d), lambda i, j: (i, 0)),
            pl.BlockSpec((1, d), lambda i, j: (0, 0)),
            pl.BlockSpec((lb, d), lambda i, j, _nj=nj: (jnp.minimum(j, _nj - 1), 0)),
            pl.BlockSpec((d, lb), lambda i, j, _nj=nj: (0, jnp.maximum(j - _nj, 0))),
        ],
        out_specs=[
            pl.BlockSpec((rb, lb), lambda i, j, _nj=nj: (i, jnp.maximum(j - _nj, 0))),
            pl.BlockSpec((rb, d), lambda i, j: (i, 0)),
        ],
        out_shape=[
            jax.ShapeDtypeStruct((tokens, latent), jnp.float32),
            jax.ShapeDtypeStruct((tokens, d), jnp.float32),
        ],
        scratch_shapes=[
            pltpu.VMEM((rb, latent), jnp.int32),
            pltpu.VMEM((rb, d), jnp.bfloat16),
            pltpu.VMEM((rb, 1), jnp.int32),
            pltpu.VMEM((rb, 1), jnp.float32),
            pltpu.VMEM((rb, 1), jnp.float32),
            pltpu.VMEM((rb, d), jnp.float32),
        ],
    )(x, b2, w_enc_bf, w_dec_bf)

    return (x_hat, z)


# speedup vs baseline: 5.0666x; 1.1132x over previous
import functools

import jax
import jax.numpy as jnp
from jax.experimental import pallas as pl
from jax.experimental.pallas import tpu as pltpu

K = 32


def _sae_kernel(x_ref, b_ref, we_ref, wd_ref, z_ref, xh_ref,
                p_ref, xn_ref, fmax_ref, thr_ref, nrm_ref, mu_ref, acc_ref,
                *, nj, lb, k):
    j = pl.program_id(1)

    @pl.when(j == 0)
    def _preprocess():
        xs = x_ref[...] - b_ref[...]
        mu = jnp.mean(xs, axis=1, keepdims=True)
        c = xs - mu
        nrm = jnp.maximum(jnp.sqrt(jnp.sum(c * c, axis=1, keepdims=True)),
                          1e-8)
        xn_ref[...] = (c / nrm).astype(jnp.bfloat16)
        nrm_ref[...] = nrm
        mu_ref[...] = mu

    @pl.when(j < nj)
    def _encode():
        p = jax.lax.dot_general(
            xn_ref[...], we_ref[...],
            dimension_numbers=(((1,), (1,)), ((), ())),
            preferred_element_type=jnp.float32)
        p_ref[:, pl.ds(j * lb, lb)] = p
        fm = jnp.where(j == 0, jnp.full_like(p, -jnp.inf), fmax_ref[...])
        fmax_ref[...] = jnp.maximum(fm, p)

    @pl.when(j == nj - 1)
    def _select():
        rb = thr_ref.shape[0]
        fm = fmax_ref[...]
        row_max = jnp.max(fm, axis=1, keepdims=True)

        def fbody(_, carry):
            lo, hi = carry
            mid = 0.5 * (lo + hi)
            cnt = jnp.sum((fm >= mid).astype(jnp.int32), axis=1,
                          keepdims=True)
            ge = cnt >= k
            return jnp.where(ge, mid, lo), jnp.where(ge, hi, mid)

        lo0 = jnp.min(fm, axis=1, keepdims=True)
        lo1, _ = jax.lax.fori_loop(0, 24, fbody, (lo0, row_max))

        hi0 = row_max + jnp.maximum(jnp.abs(row_max) * 2.0**-20, 1e-30)
        big = jnp.full((rb, 1), p_ref.shape[1], jnp.int32)

        def cond(carry):
            lo, hi, clo, chi, it = carry
            mid = 0.5 * (lo + hi)
            active = (clo - chi > 1) & (mid > lo) & (mid < hi)
            return jnp.any(active) & (it < 34)

        def body(carry):
            lo, hi, clo, chi, it = carry
            mid = 0.5 * (lo + hi)
            cnt = jnp.sum((p_ref[...] >= mid).astype(jnp.int32), axis=1,
                          keepdims=True)
            ge = cnt >= k
            return (jnp.where(ge, mid, lo), jnp.where(ge, hi, mid),
                    jnp.where(ge, cnt, clo), jnp.where(ge, chi, cnt),
                    it + 1)

        _, hi, _, _, _ = jax.lax.while_loop(
            cond, body,
            (lo1, hi0, big, jnp.zeros((rb, 1), jnp.int32), jnp.int32(0)))

        thr = jnp.max(jnp.where(p_ref[...] < hi, p_ref[...], -jnp.inf),
                      axis=1, keepdims=True)
        thr_ref[...] = jnp.maximum(thr, 0.0)

    @pl.when(j >= nj)
    def _decode():
        jj = j - nj
        p = p_ref[:, pl.ds(jj * lb, lb)]
        zblk = jnp.where(p >= thr_ref[...], jnp.maximum(p, 0.0), 0.0)
        z_ref[...] = zblk

        @pl.when(j == nj)
        def _init_acc():
            acc_ref[...] = jnp.zeros_like(acc_ref)

        acc_ref[...] += jax.lax.dot_general(
            zblk.astype(jnp.bfloat16), wd_ref[...],
            dimension_numbers=(((1,), (1,)), ((), ())),
            preferred_element_type=jnp.float32)

        @pl.when(j == 2 * nj - 1)
        def _restore():
            xh_ref[...] = acc_ref[...] * nrm_ref[...] + mu_ref[...] + b_ref[...]


@jax.jit
def kernel(x, W_enc, W_dec, b_pre):
    tokens, d = x.shape
    latent = W_enc.shape[0]
    rb = min(256, tokens)
    lb = min(1024, latent)
    ni = tokens // rb
    nj = latent // lb
    b2 = b_pre.reshape(1, d)
    w_enc_bf = W_enc.astype(jnp.bfloat16)
    w_dec_bf = W_dec.astype(jnp.bfloat16)

    z, x_hat = pl.pallas_call(
        functools.partial(_sae_kernel, nj=nj, lb=lb, k=K),
        grid=(ni, 2 * nj),
        in_specs=[
            pl.BlockSpec((rb, d), lambda i, j: (i, 0)),
            pl.BlockSpec((1, d), lambda i, j: (0, 0)),
            pl.BlockSpec((lb, d), lambda i, j, _nj=nj: (jnp.minimum(j, _nj - 1), 0)),
            pl.BlockSpec((d, lb), lambda i, j, _nj=nj: (0, jnp.maximum(j - _nj, 0))),
        ],
        out_specs=[
            pl.BlockSpec((rb, lb), lambda i, j, _nj=nj: (i, jnp.maximum(j - _nj, 0))),
            pl.BlockSpec((rb, d), lambda i, j: (i, 0)),
        ],
        out_shape=[
            jax.ShapeDtypeStruct((tokens, latent), jnp.float32),
            jax.ShapeDtypeStruct((tokens, d), jnp.float32),
        ],
        scratch_shapes=[
            pltpu.VMEM((rb, latent), jnp.float32),
            pltpu.VMEM((rb, d), jnp.bfloat16),
            pltpu.VMEM((rb, lb), jnp.float32),
            pltpu.VMEM((rb, 1), jnp.float32),
            pltpu.VMEM((rb, 1), jnp.float32),
            pltpu.VMEM((rb, 1), jnp.float32),
            pltpu.VMEM((rb, d), jnp.float32),
        ],
    )(x, b2, w_enc_bf, w_dec_bf)

    return (x_hat, z)


# speedup vs baseline: 7.8869x; 1.5567x over previous
import functools

import jax
import jax.numpy as jnp
from jax.experimental import pallas as pl
from jax.experimental.pallas import tpu as pltpu

K = 32


def _sae_kernel(x_ref, b_ref, we_ref, wd_ref, z_ref, xh_ref,
                p_ref, xn_ref, f1_ref, f2_ref, f3_ref, f4_ref,
                thr_ref, nrm_ref, mu_ref, acc_ref,
                *, nj, lb, k):
    j = pl.program_id(1)

    @pl.when(j == 0)
    def _preprocess():
        xs = x_ref[...] - b_ref[...]
        mu = jnp.mean(xs, axis=1, keepdims=True)
        c = xs - mu
        nrm = jnp.maximum(jnp.sqrt(jnp.sum(c * c, axis=1, keepdims=True)),
                          1e-8)
        xn_ref[...] = (c / nrm).astype(jnp.bfloat16)
        nrm_ref[...] = nrm
        mu_ref[...] = mu

    @pl.when(j < nj)
    def _encode():
        p = jax.lax.dot_general(
            xn_ref[...], we_ref[...],
            dimension_numbers=(((1,), (1,)), ((), ())),
            preferred_element_type=jnp.float32)
        p_ref[:, pl.ds(j * lb, lb)] = p

        @pl.when(j == 0)
        def _fold_init():
            f1_ref[...] = p
            ninf = jnp.full_like(p, -jnp.inf)
            f2_ref[...] = ninf
            f3_ref[...] = ninf
            f4_ref[...] = ninf

        @pl.when(j > 0)
        def _fold_push():
            a = f1_ref[...]
            m1 = jnp.maximum(a, p)
            t1 = jnp.minimum(a, p)
            b = f2_ref[...]
            m2 = jnp.maximum(b, t1)
            t2 = jnp.minimum(b, t1)
            c = f3_ref[...]
            m3 = jnp.maximum(c, t2)
            t3 = jnp.minimum(c, t2)
            f4_ref[...] = jnp.maximum(f4_ref[...], t3)
            f1_ref[...] = m1
            f2_ref[...] = m2
            f3_ref[...] = m3

    @pl.when(j == nj - 1)
    def _select():
        rb = thr_ref.shape[0]
        f1 = f1_ref[...]
        f2 = f2_ref[...]
        f3 = f3_ref[...]
        row_max = jnp.max(f1, axis=1, keepdims=True)
        f4max = jnp.max(f4_ref[...], axis=1, keepdims=True)
        hi0 = row_max + jnp.maximum(jnp.abs(row_max) * 2.0**-20, 1e-30)
        big = jnp.full((rb, 1), p_ref.shape[1], jnp.int32)
        zero = jnp.zeros((rb, 1), jnp.int32)

        def fcond(carry):
            lo, hi, clo, chi, it = carry
            mid = 0.5 * (lo + hi)
            active = (clo - chi > 1) & (mid > lo) & (mid < hi)
            return jnp.any(active) & (it < 64)

        def fbody(carry):
            lo, hi, clo, chi, it = carry
            mid = 0.5 * (lo + hi)
            cnt = (jnp.sum((f1 >= mid).astype(jnp.int32), axis=1,
                           keepdims=True)
                   + jnp.sum((f2 >= mid).astype(jnp.int32), axis=1,
                             keepdims=True)
                   + jnp.sum((f3 >= mid).astype(jnp.int32), axis=1,
                             keepdims=True))
            ge = cnt >= k
            return (jnp.where(ge, mid, lo), jnp.where(ge, hi, mid),
                    jnp.where(ge, cnt, clo), jnp.where(ge, chi, cnt),
                    it + 1)

        lo0 = jnp.min(f1, axis=1, keepdims=True)
        lo, hi, clo, chi, _ = jax.lax.while_loop(
            fcond, fbody, (lo0, hi0, big, zero, jnp.int32(0)))

        mid = 0.5 * (lo + hi)
        unconverged = (clo - chi > 1) & (mid > lo) & (mid < hi)
        good = (lo > f4max) & ~unconverged

        def cond(carry):
            lo, hi, clo, chi, it = carry
            mid = 0.5 * (lo + hi)
            active = (clo - chi > 1) & (mid > lo) & (mid < hi)
            return jnp.any(active) & (it < 64)

        def body(carry):
            lo, hi, clo, chi, it = carry
            mid = 0.5 * (lo + hi)
            cnt = jnp.sum((p_ref[...] >= mid).astype(jnp.int32), axis=1,
                          keepdims=True)
            ge = cnt >= k
            return (jnp.where(ge, mid, lo), jnp.where(ge, hi, mid),
                    jnp.where(ge, cnt, clo), jnp.where(ge, chi, cnt),
                    it + 1)

        lo_f, _, _, _, _ = jax.lax.while_loop(
            cond, body,
            (lo, jnp.where(good, hi, hi0),
             jnp.where(good, jnp.full((rb, 1), k, jnp.int32), big),
             jnp.where(good, jnp.full((rb, 1), k - 1, jnp.int32), zero),
             jnp.int32(0)))

        thr_ref[...] = jnp.maximum(lo_f, 0.0)

    @pl.when(j >= nj)
    def _decode():
        jj = j - nj
        p = p_ref[:, pl.ds(jj * lb, lb)]
        zblk = jnp.where(p >= thr_ref[...], jnp.maximum(p, 0.0), 0.0)
        z_ref[...] = zblk

        @pl.when(j == nj)
        def _init_acc():
            acc_ref[...] = jnp.zeros_like(acc_ref)

        acc_ref[...] += jax.lax.dot_general(
            zblk.astype(jnp.bfloat16), wd_ref[...],
            dimension_numbers=(((1,), (1,)), ((), ())),
            preferred_element_type=jnp.float32)

        @pl.when(j == 2 * nj - 1)
        def _restore():
            xh_ref[...] = acc_ref[...] * nrm_ref[...] + mu_ref[...] + b_ref[...]


@jax.jit
def kernel(x, W_enc, W_dec, b_pre):
    tokens, d = x.shape
    latent = W_enc.shape[0]
    rb = min(256, tokens)
    lb = min(1024, latent)
    ni = tokens // rb
    nj = latent // lb
    b2 = b_pre.reshape(1, d)
    w_enc_bf = W_enc.astype(jnp.bfloat16)
    w_dec_bf = W_dec.astype(jnp.bfloat16)

    z, x_hat = pl.pallas_call(
        functools.partial(_sae_kernel, nj=nj, lb=lb, k=K),
        grid=(ni, 2 * nj),
        in_specs=[
            pl.BlockSpec((rb, d), lambda i, j: (i, 0)),
            pl.BlockSpec((1, d), lambda i, j: (0, 0)),
            pl.BlockSpec((lb, d), lambda i, j, _nj=nj: (jnp.minimum(j, _nj - 1), 0)),
            pl.BlockSpec((d, lb), lambda i, j, _nj=nj: (0, jnp.maximum(j - _nj, 0))),
        ],
        out_specs=[
            pl.BlockSpec((rb, lb), lambda i, j, _nj=nj: (i, jnp.maximum(j - _nj, 0))),
            pl.BlockSpec((rb, d), lambda i, j: (i, 0)),
        ],
        out_shape=[
            jax.ShapeDtypeStruct((tokens, latent), jnp.float32),
            jax.ShapeDtypeStruct((tokens, d), jnp.float32),
        ],
        scratch_shapes=[
            pltpu.VMEM((rb, latent), jnp.float32),
            pltpu.VMEM((rb, d), jnp.bfloat16),
            pltpu.VMEM((rb, lb), jnp.float32),
            pltpu.VMEM((rb, lb), jnp.float32),
            pltpu.VMEM((rb, lb), jnp.float32),
            pltpu.VMEM((rb, lb), jnp.float32),
            pltpu.VMEM((rb, 1), jnp.float32),
            pltpu.VMEM((rb, 1), jnp.float32),
            pltpu.VMEM((rb, 1), jnp.float32),
            pltpu.VMEM((rb, d), jnp.float32),
        ],
    )(x, b2, w_enc_bf, w_dec_bf)

    return (x_hat, z)
